# baseline (speedup 1.0000x reference)
import functools

import jax
import jax.numpy as jnp
from jax import lax
from jax.experimental import pallas as pl
from jax.experimental.pallas import tpu as pltpu
from jax.experimental.pallas import tpu_sc as plsc

N = 25000
E = 312500
NC, NS = 2, 16
NW = NC * NS
CH = 128
CPW = 80
EPW = CPW * CH
E_PAD = NW * EPW
NA_PAD = 25088
RPS = NA_PAD // NS
NBUF = 4


def _make_segsum(D):
    mesh = plsc.VectorSubcoreMesh(core_axis_name="c", subcore_axis_name="s")

    @functools.partial(
        pl.kernel,
        mesh=mesh,
        compiler_params=pltpu.CompilerParams(use_tc_tiling_on_sc=False),
        out_type=jax.ShapeDtypeStruct((NC, NA_PAD, D), jnp.float32),
        scratch_types=(
            [pltpu.VMEM((EPW,), jnp.int32),
             pltpu.VMEM((EPW,), jnp.int32)]
            + [pltpu.VMEM((CH, D), jnp.float32) for _ in range(2)]
            + [pltpu.VMEM_SHARED((NA_PAD, D), jnp.float32)]
            + [pltpu.SemaphoreType.DMA for _ in range(2)]
        ),
    )
    def seg(values_h, src_h, dst_h, zeros_h, out_h, src_v, dst_v, *rest):
        rows = rest[:2]
        acc_s = rest[2]
        gsem = rest[3:5]
        cid = lax.axis_index("c")
        sid = lax.axis_index("s")
        wid = sid * NC + cid
        base = wid * EPW

        pltpu.sync_copy(src_h.at[pl.ds(base, EPW)], src_v)
        pltpu.sync_copy(dst_h.at[pl.ds(base, EPW)], dst_v)
        pltpu.sync_copy(zeros_h.at[pl.ds(sid * RPS, RPS)],
                        acc_s.at[pl.ds(sid * RPS, RPS)])
        plsc.subcore_barrier()

        def gather(jj, b):
            pltpu.async_copy(values_h.at[src_v.at[pl.ds(jj * CH, CH)]],
                             rows[b], gsem[b])

        def gather_wait(jj, b):
            pltpu.make_async_copy(values_h.at[src_v.at[pl.ds(jj * CH, CH)]],
                                  rows[b], gsem[b]).wait()

        def scatter(jj, b):
            pltpu.sync_copy(rows[b], acc_s.at[dst_v.at[pl.ds(jj * CH, CH)]],
                            add=True)

        gather(0, 0)

        def body(i, carry):
            j0 = i * 2
            gather(j0 + 1, 1)
            gather_wait(j0, 0)
            scatter(j0, 0)

            @pl.when(j0 + 2 < CPW)
            def _():
                gather(j0 + 2, 0)

            gather_wait(j0 + 1, 1)
            scatter(j0 + 1, 1)
            return carry

        lax.fori_loop(0, CPW // 2, body, 0, unroll=False)
        plsc.subcore_barrier()

        pltpu.sync_copy(acc_s.at[pl.ds(sid * RPS, RPS)],
                        out_h.at[cid, pl.ds(sid * RPS, RPS)])

    return seg


_SEG = {}


def _segsum(D, *args):
    if D not in _SEG:
        _SEG[D] = _make_segsum(D)
    return _SEG[D](*args)


_HI = jax.lax.Precision.HIGHEST


def _stage_b(s2p, xip, A, B, bias):
    def body(s_r, x_r, a_r, b_r, c_r, o_r):
        s = s_r[0] + s_r[1]
        o_r[...] = (jnp.dot(s, a_r[...], precision=_HI)
                    + jnp.dot(x_r[...], b_r[...], precision=_HI)
                    + c_r[...])

    full = lambda shape: pl.BlockSpec(shape, lambda: tuple(0 for _ in shape))
    return pl.pallas_call(
        body,
        in_specs=[full((2, 1568, 128)), full((1568, 64)),
                  full((128, 256)), full((64, 256)), full((1, 256))],
        out_specs=full((1568, 256)),
        out_shape=jax.ShapeDtypeStruct((1568, 256), jnp.float32),
    )(s2p, xip, A, B, bias)


def _stage_d(ttp, xup, A, B, bias):
    def body(t_r, x_r, a_r, b_r, c_r, o_r):
        o = t_r[0] + t_r[1]
        o_r[...] = (jnp.dot(o, a_r[...], precision=_HI)
                    + jnp.dot(x_r[...], b_r[...], precision=_HI)
                    + c_r[...])

    full = lambda shape: pl.BlockSpec(shape, lambda: tuple(0 for _ in shape))
    return pl.pallas_call(
        body,
        in_specs=[full((2, 3136, 128)), full((3136, 32)),
                  full((128, 128)), full((32, 128)), full((1, 128))],
        out_specs=full((3136, 128)),
        out_shape=jax.ShapeDtypeStruct((3136, 128), jnp.float32),
    )(ttp, xup, A, B, bias)


def _prep_edges(ei):
    npad = E_PAD - E
    src = jnp.concatenate([ei[0], jnp.zeros((npad,), jnp.int32)])
    dst = jnp.concatenate(
        [ei[1], N + (jnp.arange(npad, dtype=jnp.int32) % (NA_PAD - N))])
    return src, dst


def _fold(We, be, Wmat):
    W3 = Wmat.reshape(4, 32, -1)
    return (jnp.einsum("ck,ckj->cj", We, W3, precision=_HI),
            jnp.einsum("ck,ckj->j", be, W3, precision=_HI))


def _mm(a, b):
    return jnp.matmul(a, b, precision=_HI)


def _pad16(a):
    a = jnp.atleast_2d(a)
    return jnp.pad(a, ((0, 0), (0, 16 - a.shape[1])))


def _blockdiag(E_node, groups):
    k = E_node.shape[0]
    eye = jnp.eye(groups, dtype=jnp.float32)
    return (eye[:, None, :, None] * E_node[None, :, None, :]).reshape(
        groups * k, groups * 16)


def _pack_cols(x, per_row):
    xp = jnp.pad(x, ((0, NA_PAD - N), (0, 0)))
    return xp.reshape(-1, per_row)


def kernel(x_user, x_item, edge_u2i, edge_i2u,
           emb_W_user, emb_b_user, emb_W_item, emb_b_item,
           Wl1_u, bl1_u, Wr1_u, Wl1_i, bl1_i, Wr1_i,
           Wl2_u, bl2_u, Wr2_u, Wl2_i, bl2_i, Wr2_i,
           Wm, bm):
    C = _mm(Wl2_u, Wm)
    G = _mm(Wr2_u, Wm)
    e = _mm(bl2_u, Wm) + bm
    Ml_i, cl_i = _fold(emb_W_user, emb_b_user, Wl1_i)
    Mr_i, cr_i = _fold(emb_W_item, emb_b_item, Wr1_i)
    P, p, Rm, r0 = _mm(Ml_i, C), _mm(cl_i, C), _mm(Mr_i, C), _mm(bl1_i + cr_i, C)
    Ml_u, cl_u = _fold(emb_W_item, emb_b_item, Wl1_u)
    Mr_u, cr_u = _fold(emb_W_user, emb_b_user, Wr1_u)
    P2, p2, R2, r2 = (_mm(Ml_u, G), _mm(cl_u, G), _mm(Mr_u, G),
                      _mm(bl1_u + cr_u, G) + e)

    E8 = jnp.concatenate([_pad16(P), _pad16(p), jnp.zeros((3, 16))], axis=0)
    RS = _pad16(Rm).at[jnp.arange(4), 10 + jnp.arange(4)].set(1.0)
    cB = _pad16(r0).at[0, 14].set(1.0)
    A_B = _blockdiag(E8, 16)
    B_B = _blockdiag(RS, 16)
    bias_B = jnp.tile(cB, (1, 16))

    E16 = jnp.eye(16, dtype=jnp.float32)
    E16 = E16.at[10:14, :].add(_pad16(P2))
    E16 = E16.at[14:15, :].add(_pad16(p2))
    cD = _pad16(r2)
    A_D = _blockdiag(E16, 8)
    B_D = _blockdiag(_pad16(R2), 8)
    bias_D = jnp.tile(cD, (1, 8))

    v1 = jnp.concatenate(
        [x_user, jnp.ones((N, 1), jnp.float32), jnp.zeros((N, 3), jnp.float32)],
        axis=1)
    srcA, dstA = _prep_edges(edge_u2i)
    z8 = jnp.zeros((NA_PAD, 8), jnp.float32)
    sA = _segsum(8, v1, srcA, dstA, z8)

    qq = _stage_b(sA.reshape(2, 1568, 128), _pack_cols(x_item, 64),
                  A_B, B_B, bias_B).reshape(NA_PAD, 16)

    srcC, dstC = _prep_edges(edge_i2u)
    z16 = jnp.zeros((NA_PAD, 16), jnp.float32)
    tt = _segsum(16, qq, srcC, dstC, z16)

    out16 = _stage_d(tt.reshape(2, 3136, 128), _pack_cols(x_user, 32),
                     A_D, B_D, bias_D).reshape(NA_PAD, 16)
    return out16[:N, :10]


# speedup vs baseline: 1.0047x; 1.0047x over previous
import functools

import jax
import jax.numpy as jnp
from jax import lax
from jax.experimental import pallas as pl
from jax.experimental.pallas import tpu as pltpu
from jax.experimental.pallas import tpu_sc as plsc

N = 25000
E = 312500
NC, NS = 2, 16
NW = NC * NS
CH = 128
CPW = 80
EPW = CPW * CH
E_PAD = NW * EPW
NA_PAD = 25088
RPS = NA_PAD // NS
NBUF = 4


def _make_segsum(D):
    mesh = plsc.VectorSubcoreMesh(core_axis_name="c", subcore_axis_name="s")

    @functools.partial(
        pl.kernel,
        mesh=mesh,
        compiler_params=pltpu.CompilerParams(use_tc_tiling_on_sc=False),
        out_type=jax.ShapeDtypeStruct((NC, NA_PAD, D), jnp.float32),
        scratch_types=(
            [pltpu.VMEM((CPW, CH), jnp.int32),
             pltpu.VMEM((CPW, CH), jnp.int32)]
            + [pltpu.VMEM((CH, D), jnp.float32) for _ in range(2)]
            + [pltpu.VMEM_SHARED((NA_PAD, D), jnp.float32)]
            + [pltpu.SemaphoreType.DMA for _ in range(2)]
        ),
    )
    def seg(values_h, src_h, dst_h, zeros_h, out_h, src_v, dst_v, *rest):
        rows = rest[:2]
        acc_s = rest[2]
        gsem = rest[3:5]
        cid = lax.axis_index("c")
        sid = lax.axis_index("s")
        wid = sid * NC + cid
        base = wid * EPW

        def stage(i, carry):
            pltpu.async_copy(src_h.at[pl.ds(base + i * CH, CH)],
                             src_v.at[i], gsem[0])
            pltpu.async_copy(dst_h.at[pl.ds(base + i * CH, CH)],
                             dst_v.at[i], gsem[1])
            return carry

        lax.fori_loop(0, CPW, stage, 0, unroll=False)

        def stage_wait(i, carry):
            pltpu.make_async_copy(src_h.at[pl.ds(base + i * CH, CH)],
                                  src_v.at[i], gsem[0]).wait()
            pltpu.make_async_copy(dst_h.at[pl.ds(base + i * CH, CH)],
                                  dst_v.at[i], gsem[1]).wait()
            return carry

        lax.fori_loop(0, CPW, stage_wait, 0, unroll=False)
        pltpu.sync_copy(zeros_h.at[pl.ds(sid * RPS, RPS)],
                        acc_s.at[pl.ds(sid * RPS, RPS)])
        plsc.subcore_barrier()

        def gather(jj, b):
            pltpu.async_copy(values_h.at[src_v.at[jj]], rows[b], gsem[b])

        def gather_wait(jj, b):
            pltpu.make_async_copy(values_h.at[src_v.at[jj]],
                                  rows[b], gsem[b]).wait()

        def scatter(jj, b):
            pltpu.sync_copy(rows[b], acc_s.at[dst_v.at[jj]], add=True)

        gather(0, 0)

        def body(i, carry):
            j0 = i * 2
            gather(j0 + 1, 1)
            gather_wait(j0, 0)
            scatter(j0, 0)

            @pl.when(j0 + 2 < CPW)
            def _():
                gather(j0 + 2, 0)

            gather_wait(j0 + 1, 1)
            scatter(j0 + 1, 1)
            return carry

        lax.fori_loop(0, CPW // 2, body, 0, unroll=False)
        plsc.subcore_barrier()

        pltpu.sync_copy(acc_s.at[pl.ds(sid * RPS, RPS)],
                        out_h.at[cid, pl.ds(sid * RPS, RPS)])

    return seg


_SEG = {}


def _segsum(D, *args):
    if D not in _SEG:
        _SEG[D] = _make_segsum(D)
    return _SEG[D](*args)


_HI = jax.lax.Precision.HIGHEST


def _stage_b(s2p, xip, A, B, bias):
    def body(s_r, x_r, a_r, b_r, c_r, o_r):
        s = s_r[0] + s_r[1]
        o_r[...] = (jnp.dot(s, a_r[...], precision=_HI)
                    + jnp.dot(x_r[...], b_r[...], precision=_HI)
                    + c_r[...])

    full = lambda shape: pl.BlockSpec(shape, lambda: tuple(0 for _ in shape))
    return pl.pallas_call(
        body,
        in_specs=[full((2, 1568, 128)), full((1568, 64)),
                  full((128, 256)), full((64, 256)), full((1, 256))],
        out_specs=full((1568, 256)),
        out_shape=jax.ShapeDtypeStruct((1568, 256), jnp.float32),
    )(s2p, xip, A, B, bias)


def _stage_d(ttp, xup, A, B, bias):
    def body(t_r, x_r, a_r, b_r, c_r, o_r):
        o = t_r[0] + t_r[1]
        o_r[...] = (jnp.dot(o, a_r[...], precision=_HI)
                    + jnp.dot(x_r[...], b_r[...], precision=_HI)
                    + c_r[...])

    full = lambda shape: pl.BlockSpec(shape, lambda: tuple(0 for _ in shape))
    return pl.pallas_call(
        body,
        in_specs=[full((2, 3136, 128)), full((3136, 32)),
                  full((128, 128)), full((32, 128)), full((1, 128))],
        out_specs=full((3136, 128)),
        out_shape=jax.ShapeDtypeStruct((3136, 128), jnp.float32),
    )(ttp, xup, A, B, bias)


def _prep_edges(ei):
    npad = E_PAD - E
    src = jnp.concatenate([ei[0], jnp.zeros((npad,), jnp.int32)])
    dst = jnp.concatenate(
        [ei[1], N + (jnp.arange(npad, dtype=jnp.int32) % (NA_PAD - N))])
    return src, dst


def _fold(We, be, Wmat):
    W3 = Wmat.reshape(4, 32, -1)
    return (jnp.einsum("ck,ckj->cj", We, W3, precision=_HI),
            jnp.einsum("ck,ckj->j", be, W3, precision=_HI))


def _mm(a, b):
    return jnp.matmul(a, b, precision=_HI)


def _pad16(a):
    a = jnp.atleast_2d(a)
    return jnp.pad(a, ((0, 0), (0, 16 - a.shape[1])))


def _blockdiag(E_node, groups):
    k = E_node.shape[0]
    eye = jnp.eye(groups, dtype=jnp.float32)
    return (eye[:, None, :, None] * E_node[None, :, None, :]).reshape(
        groups * k, groups * 16)


def _pack_cols(x, per_row):
    xp = jnp.pad(x, ((0, NA_PAD - N), (0, 0)))
    return xp.reshape(-1, per_row)


def kernel(x_user, x_item, edge_u2i, edge_i2u,
           emb_W_user, emb_b_user, emb_W_item, emb_b_item,
           Wl1_u, bl1_u, Wr1_u, Wl1_i, bl1_i, Wr1_i,
           Wl2_u, bl2_u, Wr2_u, Wl2_i, bl2_i, Wr2_i,
           Wm, bm):
    C = _mm(Wl2_u, Wm)
    G = _mm(Wr2_u, Wm)
    e = _mm(bl2_u, Wm) + bm
    Ml_i, cl_i = _fold(emb_W_user, emb_b_user, Wl1_i)
    Mr_i, cr_i = _fold(emb_W_item, emb_b_item, Wr1_i)
    P, p, Rm, r0 = _mm(Ml_i, C), _mm(cl_i, C), _mm(Mr_i, C), _mm(bl1_i + cr_i, C)
    Ml_u, cl_u = _fold(emb_W_item, emb_b_item, Wl1_u)
    Mr_u, cr_u = _fold(emb_W_user, emb_b_user, Wr1_u)
    P2, p2, R2, r2 = (_mm(Ml_u, G), _mm(cl_u, G), _mm(Mr_u, G),
                      _mm(bl1_u + cr_u, G) + e)

    E8 = jnp.concatenate([_pad16(P), _pad16(p), jnp.zeros((3, 16))], axis=0)
    RS = _pad16(Rm).at[jnp.arange(4), 10 + jnp.arange(4)].set(1.0)
    cB = _pad16(r0).at[0, 14].set(1.0)
    A_B = _blockdiag(E8, 16)
    B_B = _blockdiag(RS, 16)
    bias_B = jnp.tile(cB, (1, 16))

    E16 = jnp.eye(16, dtype=jnp.float32)
    E16 = E16.at[10:14, :].add(_pad16(P2))
    E16 = E16.at[14:15, :].add(_pad16(p2))
    cD = _pad16(r2)
    A_D = _blockdiag(E16, 8)
    B_D = _blockdiag(_pad16(R2), 8)
    bias_D = jnp.tile(cD, (1, 8))

    v1 = jnp.concatenate(
        [x_user, jnp.ones((N, 1), jnp.float32), jnp.zeros((N, 3), jnp.float32)],
        axis=1)
    srcA, dstA = _prep_edges(edge_u2i)
    z8 = jnp.zeros((NA_PAD, 8), jnp.float32)
    sA = _segsum(8, v1, srcA, dstA, z8)

    qq = _stage_b(sA.reshape(2, 1568, 128), _pack_cols(x_item, 64),
                  A_B, B_B, bias_B).reshape(NA_PAD, 16)

    srcC, dstC = _prep_edges(edge_i2u)
    z16 = jnp.zeros((NA_PAD, 16), jnp.float32)
    tt = _segsum(16, qq, srcC, dstC, z16)

    out16 = _stage_d(tt.reshape(2, 3136, 128), _pack_cols(x_user, 32),
                     A_D, B_D, bias_D).reshape(NA_PAD, 16)
    return out16[:N, :10]
